# initial kernel scaffold (unmeasured)
import jax
import jax.numpy as jnp
from jax import lax
from jax.experimental import pallas as pl
from jax.experimental.pallas import tpu as pltpu

N_DEV = 4


def kernel(A, B):
    m_per, k = A.shape
    _, n = B.shape

    def body(a_ref, b_ref, out_ref, ag_ref, send_sems, recv_sems):
        my = lax.axis_index("i")
        left = lax.rem(my + N_DEV - 1, N_DEV)
        right = lax.rem(my + 1, N_DEV)

        barrier_sem = pltpu.get_barrier_semaphore()
        for nbr in (left, right):
            pl.semaphore_signal(
                barrier_sem, inc=1,
                device_id=(nbr,), device_id_type=pl.DeviceIdType.MESH,
            )
        pl.semaphore_wait(barrier_sem, 2)

        ag_ref[0] = a_ref[...]

        def out_rows(h):
            origin = lax.rem(my - h + N_DEV, N_DEV)
            return pl.ds(origin * m_per, m_per)

        for h in range(N_DEV - 1):
            rdma = pltpu.make_async_remote_copy(
                src_ref=ag_ref.at[h],
                dst_ref=ag_ref.at[h + 1],
                send_sem=send_sems.at[h],
                recv_sem=recv_sems.at[h],
                device_id=(right,),
                device_id_type=pl.DeviceIdType.MESH,
            )
            rdma.start()
            out_ref[out_rows(h), :] = jnp.dot(
                ag_ref[h], b_ref[...], preferred_element_type=jnp.float32
            )
            rdma.wait()

        out_ref[out_rows(N_DEV - 1), :] = jnp.dot(
            ag_ref[N_DEV - 1], b_ref[...], preferred_element_type=jnp.float32
        )

    return pl.pallas_call(
        body,
        out_shape=jax.ShapeDtypeStruct((N_DEV * m_per, n), jnp.float32),
        in_specs=[
            pl.BlockSpec(memory_space=pltpu.VMEM),
            pl.BlockSpec(memory_space=pltpu.VMEM),
        ],
        out_specs=pl.BlockSpec(memory_space=pltpu.VMEM),
        scratch_shapes=[
            pltpu.VMEM((N_DEV, m_per, k), jnp.float32),
            pltpu.SemaphoreType.DMA((N_DEV - 1,)),
            pltpu.SemaphoreType.DMA((N_DEV - 1,)),
        ],
        compiler_params=pltpu.CompilerParams(
            collective_id=0,
            vmem_limit_bytes=100 * 1024 * 1024,
        ),
    )(A, B)


# baseline (device time: 206078 ns/iter reference)
import jax
import jax.numpy as jnp
from jax import lax
from jax.experimental import pallas as pl
from jax.experimental.pallas import tpu as pltpu

N_DEV = 4


def kernel(A, B):
    m_per, k = A.shape
    _, n = B.shape

    def body(a_ref, b_ref, out_ref, ag_ref, send_sems, recv_sems):
        my = lax.axis_index("i")
        left = lax.rem(my + N_DEV - 1, N_DEV)
        right = lax.rem(my + 1, N_DEV)

        barrier_sem = pltpu.get_barrier_semaphore()
        for nbr in (left, right):
            pl.semaphore_signal(
                barrier_sem, inc=1,
                device_id=(nbr,), device_id_type=pl.DeviceIdType.MESH,
            )
        pl.semaphore_wait(barrier_sem, 2)

        def out_rows(h):
            origin = lax.rem(my - h - 1 + 2 * N_DEV, N_DEV)
            return pl.ds(origin * m_per, m_per)

        for h in range(N_DEV - 1):
            rdma = pltpu.make_async_remote_copy(
                src_ref=a_ref if h == 0 else ag_ref.at[h - 1],
                dst_ref=ag_ref.at[h],
                send_sem=send_sems.at[h],
                recv_sem=recv_sems.at[h],
                device_id=(right,),
                device_id_type=pl.DeviceIdType.MESH,
            )
            rdma.start()
            out_ref[out_rows(h - 1), :] = jnp.dot(
                a_ref[...] if h == 0 else ag_ref[h - 1],
                b_ref[...],
                preferred_element_type=jnp.float32,
            )
            rdma.wait()

        out_ref[out_rows(N_DEV - 2), :] = jnp.dot(
            ag_ref[N_DEV - 2], b_ref[...], preferred_element_type=jnp.float32
        )

    return pl.pallas_call(
        body,
        out_shape=jax.ShapeDtypeStruct((N_DEV * m_per, n), jnp.float32),
        in_specs=[
            pl.BlockSpec(memory_space=pltpu.VMEM),
            pl.BlockSpec(memory_space=pltpu.VMEM),
        ],
        out_specs=pl.BlockSpec(memory_space=pltpu.VMEM),
        scratch_shapes=[
            pltpu.VMEM((N_DEV - 1, m_per, k), jnp.float32),
            pltpu.SemaphoreType.DMA((N_DEV - 1,)),
            pltpu.SemaphoreType.DMA((N_DEV - 1,)),
        ],
        compiler_params=pltpu.CompilerParams(
            collective_id=0,
            vmem_limit_bytes=100 * 1024 * 1024,
        ),
    )(A, B)


# device time: 90298 ns/iter; 2.2822x vs baseline; 2.2822x over previous
import jax
import jax.numpy as jnp
from jax import lax
from jax.experimental import pallas as pl
from jax.experimental.pallas import tpu as pltpu

N_DEV = 4


def kernel(A, B):
    m_per, k = A.shape
    _, n = B.shape
    half = m_per // 2

    def body(a_ref, b_ref, out_ref, ag_ref, bb_ref, send_sems, recv_sems):
        my = lax.axis_index("i")
        left = lax.rem(my + N_DEV - 1, N_DEV)
        right = lax.rem(my + 1, N_DEV)

        barrier_sem = pltpu.get_barrier_semaphore()
        for nbr in (left, right):
            pl.semaphore_signal(
                barrier_sem, inc=1,
                device_id=(nbr,), device_id_type=pl.DeviceIdType.MESH,
            )
        pl.semaphore_wait(barrier_sem, 2)

        def block_rows(origin):
            return pl.ds(lax.rem(origin + 2 * N_DEV, N_DEV) * m_per, m_per)

        ag_ref[0] = a_ref[...].astype(jnp.bfloat16)

        r1 = pltpu.make_async_remote_copy(
            src_ref=ag_ref.at[0], dst_ref=ag_ref.at[1],
            send_sem=send_sems.at[0], recv_sem=recv_sems.at[0],
            device_id=(right,), device_id_type=pl.DeviceIdType.MESH,
        )
        r1.start()
        l1 = pltpu.make_async_remote_copy(
            src_ref=ag_ref.at[0], dst_ref=ag_ref.at[2],
            send_sem=send_sems.at[1], recv_sem=recv_sems.at[1],
            device_id=(left,), device_id_type=pl.DeviceIdType.MESH,
        )
        l1.start()

        bb_ref[...] = b_ref[...].astype(jnp.bfloat16)
        out_ref[block_rows(my), :] = jnp.dot(
            ag_ref[0], bb_ref[...], preferred_element_type=jnp.float32
        )

        r1.wait_recv()
        r2 = pltpu.make_async_remote_copy(
            src_ref=ag_ref.at[1, pl.ds(0, half)],
            dst_ref=ag_ref.at[3, pl.ds(0, half)],
            send_sem=send_sems.at[2], recv_sem=recv_sems.at[2],
            device_id=(right,), device_id_type=pl.DeviceIdType.MESH,
        )
        r2.start()
        l1.wait_recv()
        l2 = pltpu.make_async_remote_copy(
            src_ref=ag_ref.at[2, pl.ds(half, half)],
            dst_ref=ag_ref.at[3, pl.ds(half, half)],
            send_sem=send_sems.at[3], recv_sem=recv_sems.at[3],
            device_id=(left,), device_id_type=pl.DeviceIdType.MESH,
        )
        l2.start()

        out_ref[block_rows(my - 1), :] = jnp.dot(
            ag_ref[1], bb_ref[...], preferred_element_type=jnp.float32
        )
        out_ref[block_rows(my + 1), :] = jnp.dot(
            ag_ref[2], bb_ref[...], preferred_element_type=jnp.float32
        )

        diag = lax.rem(my + 2, N_DEV)
        r2.wait_recv()
        out_ref[pl.ds(diag * m_per, half), :] = jnp.dot(
            ag_ref[3, pl.ds(0, half)], bb_ref[...],
            preferred_element_type=jnp.float32,
        )
        l2.wait_recv()
        out_ref[pl.ds(diag * m_per + half, half), :] = jnp.dot(
            ag_ref[3, pl.ds(half, half)], bb_ref[...],
            preferred_element_type=jnp.float32,
        )

        r1.wait_send()
        l1.wait_send()
        r2.wait_send()
        l2.wait_send()

    return pl.pallas_call(
        body,
        out_shape=jax.ShapeDtypeStruct((N_DEV * m_per, n), jnp.float32),
        in_specs=[
            pl.BlockSpec(memory_space=pltpu.VMEM),
            pl.BlockSpec(memory_space=pltpu.VMEM),
        ],
        out_specs=pl.BlockSpec(memory_space=pltpu.VMEM),
        scratch_shapes=[
            pltpu.VMEM((N_DEV, m_per, k), jnp.bfloat16),
            pltpu.VMEM((k, n), jnp.bfloat16),
            pltpu.SemaphoreType.DMA((4,)),
            pltpu.SemaphoreType.DMA((4,)),
        ],
        compiler_params=pltpu.CompilerParams(
            collective_id=0,
            vmem_limit_bytes=100 * 1024 * 1024,
        ),
    )(A, B)
